# baseline (speedup 1.0000x reference)
import functools

import jax
import jax.numpy as jnp
from jax import lax
from jax.experimental import pallas as pl
from jax.experimental.pallas import tpu as pltpu
from jax.experimental.pallas import tpu_sc as plsc

_N_CHUNKS = 4


def _make_lookup(B, D, b_per_w, NC, n_chunks):
    ch = b_per_w // n_chunks
    mesh = plsc.VectorSubcoreMesh(core_axis_name="c", subcore_axis_name="s")

    @functools.partial(
        pl.kernel,
        mesh=mesh,
        out_type=jax.ShapeDtypeStruct((B, D), jnp.float32),
        scratch_types=[
            pltpu.VMEM((n_chunks, ch), jnp.int32),
            pltpu.VMEM((n_chunks, ch, D), jnp.float32),
            pltpu.SemaphoreType.DMA,
            pltpu.SemaphoreType.DMA,
        ],
    )
    def lookup(table_hbm, idx_hbm, out_hbm, idx_v, rows_v, gsem, ssem):
        wid = lax.axis_index("s") * NC + lax.axis_index("c")
        base = wid * b_per_w
        pltpu.sync_copy(idx_hbm.at[wid], idx_v)
        gathers = []
        for i in range(n_chunks):
            gathers.append(
                pltpu.async_copy(table_hbm.at[idx_v.at[i]], rows_v.at[i], gsem)
            )
        scatters = []
        for i in range(n_chunks):
            gathers[i].wait()
            scatters.append(
                pltpu.async_copy(
                    rows_v.at[i], out_hbm.at[pl.ds(base + i * ch, ch)], ssem
                )
            )
        for s in scatters:
            s.wait()

    return lookup


def kernel(timestep, pos_table):
    B = timestep.shape[0]
    D = pos_table.shape[1]
    info = plsc.get_sparse_core_info()
    NC, NS = info.num_cores, info.num_subcores
    NW = NC * NS
    b_per_w = B // NW
    lookup = _make_lookup(B, D, b_per_w, NC, _N_CHUNKS)
    idx = timestep.astype(jnp.int32).reshape(NW, _N_CHUNKS, b_per_w // _N_CHUNKS)
    return lookup(pos_table, idx)


# speedup vs baseline: 1.2443x; 1.2443x over previous
import functools

import jax
import jax.numpy as jnp
from jax import lax
from jax.experimental import pallas as pl
from jax.experimental.pallas import tpu as pltpu
from jax.experimental.pallas import tpu_sc as plsc

_N_CHUNKS = 4


def _make_lookup(B, V, D, b_per_w, NC, n_chunks):
    ch = b_per_w // n_chunks
    mesh = plsc.VectorSubcoreMesh(core_axis_name="c", subcore_axis_name="s")
    n_loaders = 8
    rows_per_loader = 128

    @functools.partial(
        pl.kernel,
        mesh=mesh,
        out_type=jax.ShapeDtypeStruct((B, D), jnp.float32),
        scratch_types=[
            pltpu.VMEM_SHARED((V, D), jnp.float32),
            pltpu.VMEM((n_chunks, ch), jnp.int32),
            pltpu.VMEM((n_chunks, ch, D), jnp.float32),
            pltpu.SemaphoreType.DMA,
            pltpu.SemaphoreType.DMA,
        ],
    )
    def lookup(table_hbm, idx_hbm, out_hbm, table_s, idx_v, rows_v, gsem, ssem):
        sid = lax.axis_index("s")
        wid = sid * NC + lax.axis_index("c")
        base = wid * b_per_w
        pltpu.sync_copy(idx_hbm.at[wid], idx_v)

        for k in range(n_loaders):
            r0 = k * rows_per_loader
            nrows = min(rows_per_loader, V - r0)

            @pl.when(sid == k)
            def _load_table(r0=r0, nrows=nrows):
                pltpu.sync_copy(
                    table_hbm.at[pl.ds(r0, nrows)],
                    table_s.at[pl.ds(r0, nrows)],
                )

        plsc.subcore_barrier()
        gathers = []
        for i in range(n_chunks):
            gathers.append(
                pltpu.async_copy(table_s.at[idx_v.at[i]], rows_v.at[i], gsem)
            )
        scatters = []
        for i in range(n_chunks):
            gathers[i].wait()
            scatters.append(
                pltpu.async_copy(
                    rows_v.at[i], out_hbm.at[pl.ds(base + i * ch, ch)], ssem
                )
            )
        for s in scatters:
            s.wait()

    return lookup


def kernel(timestep, pos_table):
    B = timestep.shape[0]
    V, D = pos_table.shape
    info = plsc.get_sparse_core_info()
    NC, NS = info.num_cores, info.num_subcores
    NW = NC * NS
    b_per_w = B // NW
    lookup = _make_lookup(B, V, D, b_per_w, NC, _N_CHUNKS)
    idx = timestep.astype(jnp.int32).reshape(NW, _N_CHUNKS, b_per_w // _N_CHUNKS)
    return lookup(pos_table, idx)


# speedup vs baseline: 1.2603x; 1.0128x over previous
import functools

import jax
import jax.numpy as jnp
from jax import lax
from jax.experimental import pallas as pl
from jax.experimental.pallas import tpu as pltpu
from jax.experimental.pallas import tpu_sc as plsc

_N_CHUNKS = 8


def _make_lookup(B, V, D, b_per_w, NC, NS, n_chunks):
    ch = b_per_w // n_chunks
    mesh = plsc.VectorSubcoreMesh(core_axis_name="c", subcore_axis_name="s")
    rows_per_loader = 64

    @functools.partial(
        pl.kernel,
        mesh=mesh,
        out_type=jax.ShapeDtypeStruct((B, D), jnp.float32),
        scratch_types=[
            pltpu.VMEM_SHARED((V, D), jnp.float32),
            pltpu.VMEM((n_chunks, ch), jnp.int32),
            pltpu.VMEM((n_chunks, ch, D), jnp.float32),
            pltpu.SemaphoreType.DMA,
            pltpu.SemaphoreType.DMA,
            pltpu.SemaphoreType.DMA,
        ],
    )
    def lookup(table_hbm, idx_hbm, out_hbm, table_s, idx_v, rows_v, lsem, gsem, ssem):
        sid = lax.axis_index("s")
        wid = sid * NC + lax.axis_index("c")
        base = wid * b_per_w
        idx_cp = pltpu.async_copy(idx_hbm.at[wid], idx_v, lsem)
        for k in range(NS):
            r0 = k * rows_per_loader
            nrows = min(rows_per_loader, V - r0)

            @pl.when(sid == k)
            def _load_table(r0=r0, nrows=nrows):
                pltpu.async_copy(
                    table_hbm.at[pl.ds(r0, nrows)],
                    table_s.at[pl.ds(r0, nrows)],
                    lsem,
                ).wait()

        idx_cp.wait()
        plsc.subcore_barrier()
        gathers = []
        for i in range(n_chunks):
            gathers.append(
                pltpu.async_copy(table_s.at[idx_v.at[i]], rows_v.at[i], gsem)
            )
        scatters = []
        for i in range(n_chunks):
            gathers[i].wait()
            scatters.append(
                pltpu.async_copy(
                    rows_v.at[i], out_hbm.at[pl.ds(base + i * ch, ch)], ssem
                )
            )
        for s in scatters:
            s.wait()

    return lookup


def kernel(timestep, pos_table):
    B = timestep.shape[0]
    V, D = pos_table.shape
    info = plsc.get_sparse_core_info()
    NC, NS = info.num_cores, info.num_subcores
    NW = NC * NS
    b_per_w = B // NW
    lookup = _make_lookup(B, V, D, b_per_w, NC, NS, _N_CHUNKS)
    idx = timestep.astype(jnp.int32).reshape(NW, _N_CHUNKS, b_per_w // _N_CHUNKS)
    return lookup(pos_table, idx)


# speedup vs baseline: 1.2642x; 1.0031x over previous
import functools

import jax
import jax.numpy as jnp
from jax import lax
from jax.experimental import pallas as pl
from jax.experimental.pallas import tpu as pltpu
from jax.experimental.pallas import tpu_sc as plsc

_CHUNKS = (16, 32, 64, 128, 128, 96, 32, 16)


def _make_lookup(B, V, D, b_per_w, NC, NS, chunks):
    assert sum(chunks) == b_per_w
    mesh = plsc.VectorSubcoreMesh(core_axis_name="c", subcore_axis_name="s")
    rows_per_loader = 64

    @functools.partial(
        pl.kernel,
        mesh=mesh,
        out_type=jax.ShapeDtypeStruct((B, D), jnp.float32),
        scratch_types=[
            pltpu.VMEM_SHARED((V, D), jnp.float32),
            pltpu.VMEM((b_per_w,), jnp.int32),
            pltpu.VMEM((b_per_w, D), jnp.float32),
            pltpu.SemaphoreType.DMA,
            pltpu.SemaphoreType.DMA,
            pltpu.SemaphoreType.DMA,
        ],
    )
    def lookup(table_hbm, idx_hbm, out_hbm, table_s, idx_v, rows_v, lsem, gsem, ssem):
        sid = lax.axis_index("s")
        wid = sid * NC + lax.axis_index("c")
        base = wid * b_per_w
        idx_cp = pltpu.async_copy(idx_hbm.at[pl.ds(base, b_per_w)], idx_v, lsem)
        for k in range(NS):
            r0 = k * rows_per_loader
            nrows = min(rows_per_loader, V - r0)

            @pl.when(sid == k)
            def _load_table(r0=r0, nrows=nrows):
                pltpu.async_copy(
                    table_hbm.at[pl.ds(r0, nrows)],
                    table_s.at[pl.ds(r0, nrows)],
                    lsem,
                ).wait()

        idx_cp.wait()
        plsc.subcore_barrier()
        gathers = []
        off = 0
        for ch in chunks:
            gathers.append(
                pltpu.async_copy(
                    table_s.at[idx_v.at[pl.ds(off, ch)]],
                    rows_v.at[pl.ds(off, ch)],
                    gsem,
                )
            )
            off += ch
        scatters = []
        off = 0
        for i, ch in enumerate(chunks):
            gathers[i].wait()
            scatters.append(
                pltpu.async_copy(
                    rows_v.at[pl.ds(off, ch)],
                    out_hbm.at[pl.ds(base + off, ch)],
                    ssem,
                )
            )
            off += ch
        for s in scatters:
            s.wait()

    return lookup


def kernel(timestep, pos_table):
    B = timestep.shape[0]
    V, D = pos_table.shape
    info = plsc.get_sparse_core_info()
    NC, NS = info.num_cores, info.num_subcores
    NW = NC * NS
    b_per_w = B // NW
    lookup = _make_lookup(B, V, D, b_per_w, NC, NS, _CHUNKS)
    return lookup(pos_table, timestep.astype(jnp.int32))
